# Initial kernel scaffold; baseline (speedup 1.0000x reference)
import jax
import jax.numpy as jnp
from jax.experimental import pallas as pl


def kernel(h, edge_index, edge_attr, We1, be1, g1, bt1, We2, be2, Wm1, bm1, g2, bt2, Wm2, bm2, g3, bt3):
    raise NotImplementedError("write your pallas kernel here")


# speedup vs baseline: 1.9583x; 1.9583x over previous
import functools

import jax
import jax.numpy as jnp
from jax import lax
from jax.experimental import pallas as pl
from jax.experimental.pallas import tpu as pltpu
from jax.experimental.pallas import tpu_sc as plsc

N = 10000
E = 320000
D = 128
DE = 16

NC = 2
NS = 16
NW = NC * NS
CH = 128
JPW = 80
EPW = CH * JPW
E_PAD = NW * EPW
N_PAD = 10240
RPS = N_PAD // NS
RW = D

BLK = 2048
NB = E_PAD // BLK

_f32 = jnp.float32
_HI = lax.Precision.HIGHEST


def _stats_body(ea_ref, we1_ref, be1_ref, sy_ref, sy2_ref, sy_acc, sy2_acc):
    j = pl.program_id(0)

    @pl.when(j == 0)
    def _():
        sy_acc[...] = jnp.zeros_like(sy_acc)
        sy2_acc[...] = jnp.zeros_like(sy2_acc)

    a = ea_ref[...]
    y = jnp.dot(a, we1_ref[...], preferred_element_type=_f32) + be1_ref[...]
    rowid = j * BLK + lax.broadcasted_iota(jnp.int32, (BLK, 1), 0)
    y = jnp.where(rowid < E, y, 0.0)
    sy_acc[...] += jnp.sum(y, axis=0, keepdims=True)
    sy2_acc[...] += jnp.sum(y * y, axis=0, keepdims=True)

    @pl.when(j == NB - 1)
    def _():
        sy_ref[...] = sy_acc[...]
        sy2_ref[...] = sy2_acc[...]


def _edge_body(ea_ref, we1_ref, be1_ref, g1_ref, bt1_ref, we2_ref, be2_ref,
               sy_ref, sy2_ref, r_ref, m_scr, sv_scr):
    j = pl.program_id(0)

    @pl.when(j == 0)
    def _():
        m = sy_ref[...] / E
        var = sy2_ref[...] / E - m * m
        m_scr[...] = m
        sv_scr[...] = jnp.sqrt(var + 1e-5)

    a = ea_ref[...]
    y = jnp.dot(a, we1_ref[...], preferred_element_type=_f32) + be1_ref[...]
    rr = jnp.maximum((y - m_scr[...]) / sv_scr[...] * g1_ref[...]
                     + bt1_ref[...], 0.0)
    t = jnp.dot(rr, we2_ref[...], preferred_element_type=_f32) + be2_ref[...]
    rowid = j * BLK + lax.broadcasted_iota(jnp.int32, (BLK, 1), 0)
    r_ref[...] = jnp.where(rowid < E, t, 0.0)


def _combine_body(rp_ref, h_ref, out_ref):
    out_ref[...] = h_ref[...] + rp_ref[0] + rp_ref[1]


def _mlp_body(h2_ref, ap_ref, wm1_ref, bm1_ref, g2_ref, bt2_ref,
              wm2_ref, bm2_ref, g3_ref, bt3_ref, out_ref):
    pre = h2_ref[...] + ap_ref[0] + ap_ref[1]
    real = lax.broadcasted_iota(jnp.int32, (N_PAD, 1), 0) < N

    m1 = jnp.dot(pre, wm1_ref[...], preferred_element_type=_f32) + bm1_ref[...]
    s1 = jnp.sum(jnp.where(real, m1, 0.0), axis=0, keepdims=True) / N
    d1 = jnp.where(real, m1 - s1, 0.0)
    v1 = jnp.sum(d1 * d1, axis=0, keepdims=True) / N
    m1 = jnp.maximum((m1 - s1) / jnp.sqrt(v1 + 1e-5) * g2_ref[...]
                     + bt2_ref[...], 0.0)
    m1 = jnp.where(real, m1, 0.0)

    m2 = jnp.dot(m1, wm2_ref[...], preferred_element_type=_f32) + bm2_ref[...]
    s2 = jnp.sum(jnp.where(real, m2, 0.0), axis=0, keepdims=True) / N
    d2 = jnp.where(real, m2 - s2, 0.0)
    v2 = jnp.sum(d2 * d2, axis=0, keepdims=True) / N
    m2 = jnp.maximum((m2 - s2) / jnp.sqrt(v2 + 1e-5) * g3_ref[...]
                     + bt3_ref[...], 0.0)
    out_ref[...] = m2[:N, :]


def kernel(h, edge_index, edge_attr, We1, be1, g1, bt1, We2, be2,
           Wm1, bm1, g2, bt2, Wm2, bm2, g3, bt3):
    src = edge_index[0]
    dst = edge_index[1]
    pad = E_PAD - E
    src_p = jnp.concatenate([src, jnp.full((pad,), N, jnp.int32)]
                            ).reshape(NW, JPW, CH)
    dst_p = jnp.concatenate([dst, jnp.zeros((pad,), jnp.int32)]
                            ).reshape(NW, JPW, CH)
    ea_p = jnp.concatenate([edge_attr, jnp.zeros((pad, DE), _f32)], axis=0)
    h_p = jnp.concatenate([h, jnp.zeros((N_PAD - N, D), _f32)], axis=0)
    z128 = jnp.zeros((N_PAD, D), _f32)
    row = lambda v: v.reshape(1, D)

    sy, sy2 = pl.pallas_call(
        _stats_body,
        grid=(NB,),
        in_specs=[pl.BlockSpec((BLK, DE), lambda j: (j, 0)),
                  pl.BlockSpec((DE, D), lambda j: (0, 0)),
                  pl.BlockSpec((1, D), lambda j: (0, 0))],
        out_specs=[pl.BlockSpec((1, D), lambda j: (0, 0)),
                   pl.BlockSpec((1, D), lambda j: (0, 0))],
        out_shape=[jax.ShapeDtypeStruct((1, D), _f32),
                   jax.ShapeDtypeStruct((1, D), _f32)],
        scratch_shapes=[pltpu.VMEM((1, D), _f32), pltpu.VMEM((1, D), _f32)],
    )(ea_p, We1, row(be1))

    r_full = pl.pallas_call(
        _edge_body,
        grid=(NB,),
        in_specs=[pl.BlockSpec((BLK, DE), lambda j: (j, 0)),
                  pl.BlockSpec((DE, D), lambda j: (0, 0)),
                  pl.BlockSpec((1, D), lambda j: (0, 0)),
                  pl.BlockSpec((1, D), lambda j: (0, 0)),
                  pl.BlockSpec((1, D), lambda j: (0, 0)),
                  pl.BlockSpec((D, D), lambda j: (0, 0)),
                  pl.BlockSpec((1, D), lambda j: (0, 0)),
                  pl.BlockSpec((1, D), lambda j: (0, 0)),
                  pl.BlockSpec((1, D), lambda j: (0, 0))],
        out_specs=pl.BlockSpec((BLK, RW), lambda j: (j, 0)),
        out_shape=jax.ShapeDtypeStruct((E_PAD, RW), _f32),
        scratch_shapes=[pltpu.VMEM((1, D), _f32), pltpu.VMEM((1, D), _f32)],
    )(ea_p, We1, row(be1), row(g1), row(bt1), We2, row(be2), sy, sy2)

    mesh = plsc.VectorSubcoreMesh(core_axis_name="c", subcore_axis_name="s",
                                  num_cores=NC, num_subcores=NS)

    @functools.partial(
        pl.kernel, mesh=mesh,
        out_type=jax.ShapeDtypeStruct((NC, N_PAD, RW), _f32),
        scratch_types=[pltpu.VMEM((CH,), jnp.int32),
                       pltpu.VMEM((CH, RW), _f32),
                       pltpu.VMEM_SHARED((N_PAD, RW), _f32)],
    )
    def scatter_r(r_hbm, dst_hbm, z_hbm, out_hbm, idx_v, rows_v, acc_sh):
        c = lax.axis_index("c")
        s = lax.axis_index("s")
        w = s * NC + c
        pltpu.sync_copy(z_hbm.at[pl.ds(s * RPS, RPS)],
                        acc_sh.at[pl.ds(s * RPS, RPS)])
        plsc.subcore_barrier()

        @pl.loop(0, JPW)
        def _(j):
            base = w * EPW + j * CH
            pltpu.sync_copy(dst_hbm.at[w, j], idx_v)
            pltpu.sync_copy(r_hbm.at[pl.ds(base, CH)], rows_v)
            pltpu.sync_copy(rows_v, acc_sh.at[idx_v], add=True)

        plsc.subcore_barrier()
        pltpu.sync_copy(acc_sh.at[pl.ds(s * RPS, RPS)],
                        out_hbm.at[c].at[pl.ds(s * RPS, RPS)])

    rp = scatter_r(r_full, dst_p, z128)

    h2 = pl.pallas_call(
        _combine_body,
        out_shape=jax.ShapeDtypeStruct((N_PAD, D), _f32),
    )(rp, h_p)

    @functools.partial(
        pl.kernel, mesh=mesh,
        out_type=jax.ShapeDtypeStruct((NC, N_PAD, D), _f32),
        scratch_types=[pltpu.VMEM((CH,), jnp.int32),
                       pltpu.VMEM((CH,), jnp.int32),
                       pltpu.VMEM((CH, D), _f32),
                       pltpu.VMEM_SHARED((N_PAD, D), _f32),
                       pltpu.SemaphoreType.DMA],
    )
    def agg_k(h2_hbm, src_hbm, dst_hbm, z_hbm, out_hbm,
              si_v, di_v, rows_v, acc_sh, sem):
        c = lax.axis_index("c")
        s = lax.axis_index("s")
        w = s * NC + c
        pltpu.sync_copy(z_hbm.at[pl.ds(s * RPS, RPS)],
                        acc_sh.at[pl.ds(s * RPS, RPS)])
        plsc.subcore_barrier()

        @pl.loop(0, JPW)
        def _(j):
            pltpu.sync_copy(src_hbm.at[w, j], si_v)
            pltpu.sync_copy(dst_hbm.at[w, j], di_v)
            pltpu.async_copy(h2_hbm.at[si_v], rows_v, sem).wait()
            pltpu.sync_copy(rows_v, acc_sh.at[di_v], add=True)

        plsc.subcore_barrier()
        pltpu.sync_copy(acc_sh.at[pl.ds(s * RPS, RPS)],
                        out_hbm.at[c].at[pl.ds(s * RPS, RPS)])

    ap = agg_k(h2, src_p, dst_p, z128)

    out = pl.pallas_call(
        _mlp_body,
        out_shape=jax.ShapeDtypeStruct((N, D), _f32),
    )(h2, ap, Wm1, row(bm1), row(g2), row(bt2), Wm2, row(bm2), row(g3),
      row(bt3))
    return out


# speedup vs baseline: 2.3161x; 1.1827x over previous
import functools

import jax
import jax.numpy as jnp
from jax import lax
from jax.experimental import pallas as pl
from jax.experimental.pallas import tpu as pltpu
from jax.experimental.pallas import tpu_sc as plsc

N = 10000
E = 320000
D = 128
DE = 16

NC = 2
NS = 16
NW = NC * NS
CH = 128
JPW = 80
EPW = CH * JPW
E_PAD = NW * EPW
N_PAD = 10240
RPS = N_PAD // NS
RW = D

BLK = 2048
NB = E_PAD // BLK

_f32 = jnp.float32
_HI = lax.Precision.HIGHEST


def _stats_body(ea_ref, we1_ref, be1_ref, sy_ref, sy2_ref, sy_acc, sy2_acc):
    j = pl.program_id(0)

    @pl.when(j == 0)
    def _():
        sy_acc[...] = jnp.zeros_like(sy_acc)
        sy2_acc[...] = jnp.zeros_like(sy2_acc)

    a = ea_ref[...]
    y = jnp.dot(a, we1_ref[...], preferred_element_type=_f32) + be1_ref[...]
    rowid = j * BLK + lax.broadcasted_iota(jnp.int32, (BLK, 1), 0)
    y = jnp.where(rowid < E, y, 0.0)
    sy_acc[...] += jnp.sum(y, axis=0, keepdims=True)
    sy2_acc[...] += jnp.sum(y * y, axis=0, keepdims=True)

    @pl.when(j == NB - 1)
    def _():
        sy_ref[...] = sy_acc[...]
        sy2_ref[...] = sy2_acc[...]


def _edge_body(ea_ref, we1_ref, be1_ref, g1_ref, bt1_ref, we2_ref, be2_ref,
               sy_ref, sy2_ref, r_ref, m_scr, sv_scr):
    j = pl.program_id(0)

    @pl.when(j == 0)
    def _():
        m = sy_ref[...] / E
        var = sy2_ref[...] / E - m * m
        m_scr[...] = m
        sv_scr[...] = jnp.sqrt(var + 1e-5)

    a = ea_ref[...]
    y = jnp.dot(a, we1_ref[...], preferred_element_type=_f32) + be1_ref[...]
    rr = jnp.maximum((y - m_scr[...]) / sv_scr[...] * g1_ref[...]
                     + bt1_ref[...], 0.0)
    t = jnp.dot(rr, we2_ref[...], preferred_element_type=_f32) + be2_ref[...]
    rowid = j * BLK + lax.broadcasted_iota(jnp.int32, (BLK, 1), 0)
    r_ref[...] = jnp.where(rowid < E, t, 0.0)


def _combine_body(rp_ref, h_ref, out_ref):
    out_ref[...] = h_ref[...] + rp_ref[0] + rp_ref[1]


def _mlp_body(h2_ref, ap_ref, wm1_ref, bm1_ref, g2_ref, bt2_ref,
              wm2_ref, bm2_ref, g3_ref, bt3_ref, out_ref):
    pre = h2_ref[...] + ap_ref[0] + ap_ref[1]
    real = lax.broadcasted_iota(jnp.int32, (N_PAD, 1), 0) < N

    m1 = jnp.dot(pre, wm1_ref[...], preferred_element_type=_f32) + bm1_ref[...]
    s1 = jnp.sum(jnp.where(real, m1, 0.0), axis=0, keepdims=True) / N
    d1 = jnp.where(real, m1 - s1, 0.0)
    v1 = jnp.sum(d1 * d1, axis=0, keepdims=True) / N
    m1 = jnp.maximum((m1 - s1) / jnp.sqrt(v1 + 1e-5) * g2_ref[...]
                     + bt2_ref[...], 0.0)
    m1 = jnp.where(real, m1, 0.0)

    m2 = jnp.dot(m1, wm2_ref[...], preferred_element_type=_f32) + bm2_ref[...]
    s2 = jnp.sum(jnp.where(real, m2, 0.0), axis=0, keepdims=True) / N
    d2 = jnp.where(real, m2 - s2, 0.0)
    v2 = jnp.sum(d2 * d2, axis=0, keepdims=True) / N
    m2 = jnp.maximum((m2 - s2) / jnp.sqrt(v2 + 1e-5) * g3_ref[...]
                     + bt3_ref[...], 0.0)
    out_ref[...] = m2[:N, :]


def kernel(h, edge_index, edge_attr, We1, be1, g1, bt1, We2, be2,
           Wm1, bm1, g2, bt2, Wm2, bm2, g3, bt3):
    src = edge_index[0]
    dst = edge_index[1]
    pad = E_PAD - E
    src_p = jnp.concatenate([src, jnp.full((pad,), N, jnp.int32)]
                            ).reshape(NW, JPW, CH)
    dst_p = jnp.concatenate([dst, jnp.zeros((pad,), jnp.int32)]
                            ).reshape(NW, JPW, CH)
    ea_p = jnp.concatenate([edge_attr, jnp.zeros((pad, DE), _f32)], axis=0)
    h_p = jnp.concatenate([h, jnp.zeros((N_PAD - N, D), _f32)], axis=0)
    z128 = jnp.zeros((N_PAD, D), _f32)
    row = lambda v: v.reshape(1, D)

    sy, sy2 = pl.pallas_call(
        _stats_body,
        grid=(NB,),
        in_specs=[pl.BlockSpec((BLK, DE), lambda j: (j, 0)),
                  pl.BlockSpec((DE, D), lambda j: (0, 0)),
                  pl.BlockSpec((1, D), lambda j: (0, 0))],
        out_specs=[pl.BlockSpec((1, D), lambda j: (0, 0)),
                   pl.BlockSpec((1, D), lambda j: (0, 0))],
        out_shape=[jax.ShapeDtypeStruct((1, D), _f32),
                   jax.ShapeDtypeStruct((1, D), _f32)],
        scratch_shapes=[pltpu.VMEM((1, D), _f32), pltpu.VMEM((1, D), _f32)],
    )(ea_p, We1, row(be1))

    r_full = pl.pallas_call(
        _edge_body,
        grid=(NB,),
        in_specs=[pl.BlockSpec((BLK, DE), lambda j: (j, 0)),
                  pl.BlockSpec((DE, D), lambda j: (0, 0)),
                  pl.BlockSpec((1, D), lambda j: (0, 0)),
                  pl.BlockSpec((1, D), lambda j: (0, 0)),
                  pl.BlockSpec((1, D), lambda j: (0, 0)),
                  pl.BlockSpec((D, D), lambda j: (0, 0)),
                  pl.BlockSpec((1, D), lambda j: (0, 0)),
                  pl.BlockSpec((1, D), lambda j: (0, 0)),
                  pl.BlockSpec((1, D), lambda j: (0, 0))],
        out_specs=pl.BlockSpec((BLK, RW), lambda j: (j, 0)),
        out_shape=jax.ShapeDtypeStruct((E_PAD, RW), _f32),
        scratch_shapes=[pltpu.VMEM((1, D), _f32), pltpu.VMEM((1, D), _f32)],
    )(ea_p, We1, row(be1), row(g1), row(bt1), We2, row(be2), sy, sy2)

    mesh = plsc.VectorSubcoreMesh(core_axis_name="c", subcore_axis_name="s",
                                  num_cores=NC, num_subcores=NS)

    @functools.partial(
        pl.kernel, mesh=mesh,
        out_type=jax.ShapeDtypeStruct((NC, N_PAD, RW), _f32),
        scratch_types=[pltpu.VMEM((2, CH), jnp.int32),
                       pltpu.VMEM((2, CH, RW), _f32),
                       pltpu.VMEM_SHARED((N_PAD, RW), _f32),
                       pltpu.SemaphoreType.DMA,
                       pltpu.SemaphoreType.DMA],
    )
    def scatter_r(r_hbm, dst_hbm, z_hbm, out_hbm, idx_v, rows_v, acc_sh,
                  sem0, sem1):
        c = lax.axis_index("c")
        s = lax.axis_index("s")
        w = s * NC + c
        base = w * EPW
        pltpu.sync_copy(z_hbm.at[pl.ds(s * RPS, RPS)],
                        acc_sh.at[pl.ds(s * RPS, RPS)])
        plsc.subcore_barrier()

        def start(j, b, sem):
            pltpu.sync_copy(dst_hbm.at[w, j], idx_v.at[b])
            pltpu.async_copy(r_hbm.at[pl.ds(base + j * CH, CH)],
                             rows_v.at[b], sem)

        def finish(b, sem):
            pltpu.make_async_copy(r_hbm.at[pl.ds(base, CH)],
                                  rows_v.at[b], sem).wait()
            pltpu.sync_copy(rows_v.at[b], acc_sh.at[idx_v.at[b]], add=True)

        start(0, 0, sem0)

        @pl.loop(0, JPW // 2)
        def _(i):
            j = i * 2
            start(j + 1, 1, sem1)
            finish(0, sem0)

            @pl.when(j + 2 < JPW)
            def _():
                start(j + 2, 0, sem0)

            finish(1, sem1)

        plsc.subcore_barrier()
        pltpu.sync_copy(acc_sh.at[pl.ds(s * RPS, RPS)],
                        out_hbm.at[c].at[pl.ds(s * RPS, RPS)])

    rp = scatter_r(r_full, dst_p, z128)

    h2 = pl.pallas_call(
        _combine_body,
        out_shape=jax.ShapeDtypeStruct((N_PAD, D), _f32),
    )(rp, h_p)

    @functools.partial(
        pl.kernel, mesh=mesh,
        out_type=jax.ShapeDtypeStruct((NC, N_PAD, D), _f32),
        scratch_types=[pltpu.VMEM((2, CH), jnp.int32),
                       pltpu.VMEM((2, CH), jnp.int32),
                       pltpu.VMEM((2, CH, D), _f32),
                       pltpu.VMEM_SHARED((N_PAD, D), _f32),
                       pltpu.SemaphoreType.DMA,
                       pltpu.SemaphoreType.DMA],
    )
    def agg_k(h2_hbm, src_hbm, dst_hbm, z_hbm, out_hbm,
              si_v, di_v, rows_v, acc_sh, sem0, sem1):
        c = lax.axis_index("c")
        s = lax.axis_index("s")
        w = s * NC + c
        pltpu.sync_copy(z_hbm.at[pl.ds(s * RPS, RPS)],
                        acc_sh.at[pl.ds(s * RPS, RPS)])
        plsc.subcore_barrier()

        def start(j, b, sem):
            pltpu.sync_copy(src_hbm.at[w, j], si_v.at[b])
            pltpu.sync_copy(dst_hbm.at[w, j], di_v.at[b])
            pltpu.async_copy(h2_hbm.at[si_v.at[b]], rows_v.at[b], sem)

        def finish(b, sem):
            pltpu.make_async_copy(h2_hbm.at[si_v.at[b]],
                                  rows_v.at[b], sem).wait()
            pltpu.sync_copy(rows_v.at[b], acc_sh.at[di_v.at[b]], add=True)

        start(0, 0, sem0)

        @pl.loop(0, JPW // 2)
        def _(i):
            j = i * 2
            start(j + 1, 1, sem1)
            finish(0, sem0)

            @pl.when(j + 2 < JPW)
            def _():
                start(j + 2, 0, sem0)

            finish(1, sem1)

        plsc.subcore_barrier()
        pltpu.sync_copy(acc_sh.at[pl.ds(s * RPS, RPS)],
                        out_hbm.at[c].at[pl.ds(s * RPS, RPS)])

    ap = agg_k(h2, src_p, dst_p, z128)

    out = pl.pallas_call(
        _mlp_body,
        out_shape=jax.ShapeDtypeStruct((N, D), _f32),
    )(h2, ap, Wm1, row(bm1), row(g2), row(bt2), Wm2, row(bm2), row(g3),
      row(bt3))
    return out


# speedup vs baseline: 2.3489x; 1.0141x over previous
import functools

import jax
import jax.numpy as jnp
from jax import lax
from jax.experimental import pallas as pl
from jax.experimental.pallas import tpu as pltpu
from jax.experimental.pallas import tpu_sc as plsc

N = 10000
E = 320000
D = 128
DE = 16

NC = 2
NS = 16
NW = NC * NS
CH = 128
JPW = 80
EPW = CH * JPW
E_PAD = NW * EPW
N_PAD = 10240
RPS = N_PAD // NS
RW = D

BLK = 2048
NB = E_PAD // BLK

_f32 = jnp.float32
_HI = lax.Precision.HIGHEST


def _stats_body(ea_ref, we1_ref, be1_ref, sy_ref, sy2_ref, sy_acc, sy2_acc):
    j = pl.program_id(0)

    @pl.when(j == 0)
    def _():
        sy_acc[...] = jnp.zeros_like(sy_acc)
        sy2_acc[...] = jnp.zeros_like(sy2_acc)

    a = ea_ref[...]
    y = jnp.dot(a, we1_ref[...], preferred_element_type=_f32) + be1_ref[...]
    rowid = j * BLK + lax.broadcasted_iota(jnp.int32, (BLK, 1), 0)
    y = jnp.where(rowid < E, y, 0.0)
    sy_acc[...] += jnp.sum(y, axis=0, keepdims=True)
    sy2_acc[...] += jnp.sum(y * y, axis=0, keepdims=True)

    @pl.when(j == NB - 1)
    def _():
        sy_ref[...] = sy_acc[...]
        sy2_ref[...] = sy2_acc[...]


def _edge_body(ea_ref, we1_ref, be1_ref, g1_ref, bt1_ref, we2_ref, be2_ref,
               sy_ref, sy2_ref, r_ref, m_scr, sv_scr):
    j = pl.program_id(0)

    @pl.when(j == 0)
    def _():
        m = sy_ref[...] / E
        var = sy2_ref[...] / E - m * m
        m_scr[...] = m
        sv_scr[...] = jnp.sqrt(var + 1e-5)

    a = ea_ref[...]
    y = jnp.dot(a, we1_ref[...], preferred_element_type=_f32) + be1_ref[...]
    rr = jnp.maximum((y - m_scr[...]) / sv_scr[...] * g1_ref[...]
                     + bt1_ref[...], 0.0)
    t = jnp.dot(rr, we2_ref[...], preferred_element_type=_f32) + be2_ref[...]
    rowid = j * BLK + lax.broadcasted_iota(jnp.int32, (BLK, 1), 0)
    r_ref[...] = jnp.where(rowid < E, t, 0.0)


def _combine_body(rp_ref, h_ref, out_ref):
    out_ref[...] = h_ref[...] + rp_ref[0] + rp_ref[1]


def _mlp_body(h2_ref, ap_ref, wm1_ref, bm1_ref, g2_ref, bt2_ref,
              wm2_ref, bm2_ref, g3_ref, bt3_ref, out_ref):
    pre = h2_ref[...] + ap_ref[0] + ap_ref[1]
    real = lax.broadcasted_iota(jnp.int32, (N_PAD, 1), 0) < N

    m1 = jnp.dot(pre, wm1_ref[...], preferred_element_type=_f32) + bm1_ref[...]
    s1 = jnp.sum(jnp.where(real, m1, 0.0), axis=0, keepdims=True) / N
    d1 = jnp.where(real, m1 - s1, 0.0)
    v1 = jnp.sum(d1 * d1, axis=0, keepdims=True) / N
    m1 = jnp.maximum((m1 - s1) / jnp.sqrt(v1 + 1e-5) * g2_ref[...]
                     + bt2_ref[...], 0.0)
    m1 = jnp.where(real, m1, 0.0)

    m2 = jnp.dot(m1, wm2_ref[...], preferred_element_type=_f32) + bm2_ref[...]
    s2 = jnp.sum(jnp.where(real, m2, 0.0), axis=0, keepdims=True) / N
    d2 = jnp.where(real, m2 - s2, 0.0)
    v2 = jnp.sum(d2 * d2, axis=0, keepdims=True) / N
    m2 = jnp.maximum((m2 - s2) / jnp.sqrt(v2 + 1e-5) * g3_ref[...]
                     + bt3_ref[...], 0.0)
    out_ref[...] = m2[:N, :]


def kernel(h, edge_index, edge_attr, We1, be1, g1, bt1, We2, be2,
           Wm1, bm1, g2, bt2, Wm2, bm2, g3, bt3):
    src = edge_index[0]
    dst = edge_index[1]
    pad = E_PAD - E
    src_p = jnp.concatenate([src, jnp.full((pad,), N, jnp.int32)]
                            ).reshape(NW, JPW, CH)
    dst_p = jnp.concatenate([dst, jnp.zeros((pad,), jnp.int32)]
                            ).reshape(NW, JPW, CH)
    ea_p = jnp.concatenate([edge_attr, jnp.zeros((pad, DE), _f32)], axis=0)
    h_p = jnp.concatenate([h, jnp.zeros((N_PAD - N, D), _f32)], axis=0)
    z128 = jnp.zeros((N_PAD, D), _f32)
    row = lambda v: v.reshape(1, D)

    sy, sy2 = pl.pallas_call(
        _stats_body,
        grid=(NB,),
        in_specs=[pl.BlockSpec((BLK, DE), lambda j: (j, 0)),
                  pl.BlockSpec((DE, D), lambda j: (0, 0)),
                  pl.BlockSpec((1, D), lambda j: (0, 0))],
        out_specs=[pl.BlockSpec((1, D), lambda j: (0, 0)),
                   pl.BlockSpec((1, D), lambda j: (0, 0))],
        out_shape=[jax.ShapeDtypeStruct((1, D), _f32),
                   jax.ShapeDtypeStruct((1, D), _f32)],
        scratch_shapes=[pltpu.VMEM((1, D), _f32), pltpu.VMEM((1, D), _f32)],
    )(ea_p, We1, row(be1))

    r_full = pl.pallas_call(
        _edge_body,
        grid=(NB,),
        in_specs=[pl.BlockSpec((BLK, DE), lambda j: (j, 0)),
                  pl.BlockSpec((DE, D), lambda j: (0, 0)),
                  pl.BlockSpec((1, D), lambda j: (0, 0)),
                  pl.BlockSpec((1, D), lambda j: (0, 0)),
                  pl.BlockSpec((1, D), lambda j: (0, 0)),
                  pl.BlockSpec((D, D), lambda j: (0, 0)),
                  pl.BlockSpec((1, D), lambda j: (0, 0)),
                  pl.BlockSpec((1, D), lambda j: (0, 0)),
                  pl.BlockSpec((1, D), lambda j: (0, 0))],
        out_specs=pl.BlockSpec((BLK, RW), lambda j: (j, 0)),
        out_shape=jax.ShapeDtypeStruct((E_PAD, RW), _f32),
        scratch_shapes=[pltpu.VMEM((1, D), _f32), pltpu.VMEM((1, D), _f32)],
    )(ea_p, We1, row(be1), row(g1), row(bt1), We2, row(be2), sy, sy2)

    mesh = plsc.VectorSubcoreMesh(core_axis_name="c", subcore_axis_name="s",
                                  num_cores=NC, num_subcores=NS)

    @functools.partial(
        pl.kernel, mesh=mesh,
        out_type=jax.ShapeDtypeStruct((NC, N_PAD, RW), _f32),
        scratch_types=[pltpu.VMEM((JPW, CH), jnp.int32),
                       pltpu.VMEM((2, CH, RW), _f32),
                       pltpu.VMEM_SHARED((N_PAD, RW), _f32),
                       pltpu.SemaphoreType.DMA,
                       pltpu.SemaphoreType.DMA],
    )
    def scatter_r(r_hbm, dst_hbm, z_hbm, out_hbm, idx_v, rows_v, acc_sh,
                  sem0, sem1):
        c = lax.axis_index("c")
        s = lax.axis_index("s")
        w = s * NC + c
        base = w * EPW
        pltpu.sync_copy(dst_hbm.at[w], idx_v)
        pltpu.sync_copy(z_hbm.at[pl.ds(s * RPS, RPS)],
                        acc_sh.at[pl.ds(s * RPS, RPS)])
        plsc.subcore_barrier()

        def start(j, b, sem):
            pltpu.async_copy(r_hbm.at[pl.ds(base + j * CH, CH)],
                             rows_v.at[b], sem)

        def finish(j, b, sem):
            pltpu.make_async_copy(r_hbm.at[pl.ds(base, CH)],
                                  rows_v.at[b], sem).wait()
            pltpu.sync_copy(rows_v.at[b], acc_sh.at[idx_v.at[j]], add=True)

        start(0, 0, sem0)

        @pl.loop(0, JPW // 2)
        def _(i):
            j = i * 2
            start(j + 1, 1, sem1)
            finish(j, 0, sem0)

            @pl.when(j + 2 < JPW)
            def _():
                start(j + 2, 0, sem0)

            finish(j + 1, 1, sem1)

        plsc.subcore_barrier()
        pltpu.sync_copy(acc_sh.at[pl.ds(s * RPS, RPS)],
                        out_hbm.at[c].at[pl.ds(s * RPS, RPS)])

    rp = scatter_r(r_full, dst_p, z128)

    h2 = pl.pallas_call(
        _combine_body,
        out_shape=jax.ShapeDtypeStruct((N_PAD, D), _f32),
    )(rp, h_p)

    @functools.partial(
        pl.kernel, mesh=mesh,
        out_type=jax.ShapeDtypeStruct((NC, N_PAD, D), _f32),
        scratch_types=[pltpu.VMEM((JPW // 2, CH), jnp.int32),
                       pltpu.VMEM((JPW // 2, CH), jnp.int32),
                       pltpu.VMEM((2, CH, D), _f32),
                       pltpu.VMEM_SHARED((N_PAD, D), _f32),
                       pltpu.SemaphoreType.DMA,
                       pltpu.SemaphoreType.DMA],
    )
    def agg_k(h2_hbm, src_hbm, dst_hbm, z_hbm, out_hbm,
              si_v, di_v, rows_v, acc_sh, sem0, sem1):
        c = lax.axis_index("c")
        s = lax.axis_index("s")
        w = s * NC + c
        HJ = JPW // 2
        pltpu.sync_copy(z_hbm.at[pl.ds(s * RPS, RPS)],
                        acc_sh.at[pl.ds(s * RPS, RPS)])
        plsc.subcore_barrier()

        def start(j, b, sem):
            pltpu.async_copy(h2_hbm.at[si_v.at[j]], rows_v.at[b], sem)

        def finish(j, b, sem):
            pltpu.make_async_copy(h2_hbm.at[si_v.at[j]],
                                  rows_v.at[b], sem).wait()
            pltpu.sync_copy(rows_v.at[b], acc_sh.at[di_v.at[j]], add=True)

        for p in range(2):
            pltpu.sync_copy(src_hbm.at[w].at[pl.ds(p * HJ, HJ)], si_v)
            pltpu.sync_copy(dst_hbm.at[w].at[pl.ds(p * HJ, HJ)], di_v)
            start(0, 0, sem0)

            @pl.loop(0, HJ // 2)
            def _(i):
                j = i * 2
                start(j + 1, 1, sem1)
                finish(j, 0, sem0)

                @pl.when(j + 2 < HJ)
                def _():
                    start(j + 2, 0, sem0)

                finish(j + 1, 1, sem1)

        plsc.subcore_barrier()
        pltpu.sync_copy(acc_sh.at[pl.ds(s * RPS, RPS)],
                        out_hbm.at[c].at[pl.ds(s * RPS, RPS)])

    ap = agg_k(h2, src_p, dst_p, z128)

    out = pl.pallas_call(
        _mlp_body,
        out_shape=jax.ShapeDtypeStruct((N, D), _f32),
    )(h2, ap, Wm1, row(bm1), row(g2), row(bt2), Wm2, row(bm2), row(g3),
      row(bt3))
    return out
